# baseline (device time: 24923 ns/iter reference)
import jax
import jax.numpy as jnp
from jax import lax
from jax.experimental import pallas as pl
from jax.experimental.pallas import tpu as pltpu

N_DEV = 4
NCHUNK = 8


def kernel(x, Wp):
    b, h, w, c = x.shape
    cout = Wp.shape[1]
    hc = h // NCHUNK
    n_global = (h * N_DEV) * w
    xt = jnp.transpose(x, (0, 1, 3, 2))

    def stats_body(x_ref, st_ref, stats_ref, send_sems, recv_sems):
        my = lax.axis_index("i")

        barrier_sem = pltpu.get_barrier_semaphore()
        for d in (1, 2, 3):
            pl.semaphore_signal(
                barrier_sem,
                inc=1,
                device_id=((my + d) % N_DEV,),
                device_id_type=pl.DeviceIdType.MESH,
            )

        for bi in range(b):
            v = x_ref[bi]
            s_cw = jnp.sum(v, axis=0)
            sq_cw = jnp.sum(v * v, axis=0)
            stats_ref[N_DEV - 1, 2 * bi] = jnp.sum(s_cw, axis=1, keepdims=True)
            stats_ref[N_DEV - 1, 2 * bi + 1] = jnp.sum(
                sq_cw, axis=1, keepdims=True
            )

        pl.semaphore_wait(barrier_sem, N_DEV - 1)
        sends = []
        for d in (1, 2, 3):
            rdma = pltpu.make_async_remote_copy(
                src_ref=stats_ref.at[N_DEV - 1],
                dst_ref=stats_ref.at[d - 1],
                send_sem=send_sems.at[d - 1],
                recv_sem=recv_sems.at[d - 1],
                device_id=((my + d) % N_DEV,),
                device_id_type=pl.DeviceIdType.MESH,
            )
            rdma.start()
            sends.append(rdma)
        for d in (1, 2, 3):
            recv = pltpu.make_async_remote_copy(
                src_ref=stats_ref.at[N_DEV - 1],
                dst_ref=stats_ref.at[d - 1],
                send_sem=send_sems.at[d - 1],
                recv_sem=recv_sems.at[d - 1],
                device_id=((my + d) % N_DEV,),
                device_id_type=pl.DeviceIdType.MESH,
            )
            recv.wait_recv()
        for rdma in sends:
            rdma.wait_send()

        eps = 1e-5
        inv_n = 1.0 / float(n_global)
        for bi in range(b):
            ssum = (
                stats_ref[0, 2 * bi]
                + stats_ref[1, 2 * bi]
                + stats_ref[2, 2 * bi]
                + stats_ref[3, 2 * bi]
            )
            ssq = (
                stats_ref[0, 2 * bi + 1]
                + stats_ref[1, 2 * bi + 1]
                + stats_ref[2, 2 * bi + 1]
                + stats_ref[3, 2 * bi + 1]
            )
            mean = ssum * inv_n
            var = ssq * inv_n - mean * mean
            st_ref[2 * bi] = mean
            st_ref[2 * bi + 1] = lax.rsqrt(var + eps)

    st = pl.pallas_call(
        stats_body,
        out_shape=jax.ShapeDtypeStruct((2 * b, c, 1), jnp.float32),
        in_specs=[pl.BlockSpec(memory_space=pltpu.MemorySpace.VMEM)],
        out_specs=pl.BlockSpec(memory_space=pltpu.MemorySpace.VMEM),
        scratch_shapes=[
            pltpu.VMEM((N_DEV, 2 * b, c, 1), jnp.float32),
            pltpu.SemaphoreType.DMA((N_DEV - 1,)),
            pltpu.SemaphoreType.DMA((N_DEV - 1,)),
        ],
        compiler_params=pltpu.CompilerParams(collective_id=0),
    )(xt)

    def main_body(x_ref, wp_ref, st_ref, out_hbm, outbuf, out_sems):
        wp = wp_ref[:, :]
        out_waits = [None, None]
        k = 0
        for bi in range(b):
            mean = st_ref[2 * bi][None, :, :]
            scale = st_ref[2 * bi + 1][None, :, :]
            for ci in range(NCHUNK):
                slot = k % 2
                if out_waits[slot] is not None:
                    out_waits[slot].wait()
                v = x_ref[bi, pl.ds(ci * hc, hc), :, :]
                hh = (v - mean) * scale
                a = hh * jax.nn.sigmoid(hh)
                res = lax.dot_general(
                    a,
                    wp,
                    dimension_numbers=(((1,), (0,)), ((), ())),
                    preferred_element_type=jnp.float32,
                )
                outbuf[slot] = res.reshape(hc * w, cout)
                cp = pltpu.make_async_copy(
                    outbuf.at[slot],
                    out_hbm.at[bi, pl.ds(ci * hc * w, hc * w), :],
                    out_sems.at[slot],
                )
                cp.start()
                out_waits[slot] = cp
                k += 1
        out_waits[0].wait()
        out_waits[1].wait()

    out = pl.pallas_call(
        main_body,
        out_shape=jax.ShapeDtypeStruct((b, h * w, cout), jnp.float32),
        in_specs=[
            pl.BlockSpec(memory_space=pltpu.MemorySpace.VMEM),
            pl.BlockSpec(memory_space=pltpu.MemorySpace.VMEM),
            pl.BlockSpec(memory_space=pltpu.MemorySpace.VMEM),
        ],
        out_specs=pl.BlockSpec(memory_space=pltpu.MemorySpace.HBM),
        scratch_shapes=[
            pltpu.VMEM((2, hc * w, cout), jnp.float32),
            pltpu.SemaphoreType.DMA((2,)),
        ],
    )(xt, Wp, st)
    return out.reshape(b, h, w, cout)


# device time: 21291 ns/iter; 1.1706x vs baseline; 1.1706x over previous
import jax
import jax.numpy as jnp
from jax import lax
from jax.experimental import pallas as pl
from jax.experimental.pallas import tpu as pltpu

N_DEV = 4
NCHUNK = 2


def kernel(x, Wp):
    b, h, w, c = x.shape
    cout = Wp.shape[1]
    hc = h // NCHUNK
    n_global = (h * N_DEV) * w
    xt = jnp.transpose(x, (0, 1, 3, 2))

    def stats_body(x_ref, st_ref, stats_ref, send_sems, recv_sems):
        my = lax.axis_index("i")

        barrier_sem = pltpu.get_barrier_semaphore()
        for d in (1, 2, 3):
            pl.semaphore_signal(
                barrier_sem,
                inc=1,
                device_id=((my + d) % N_DEV,),
                device_id_type=pl.DeviceIdType.MESH,
            )

        for bi in range(b):
            v = x_ref[bi]
            s_cw = jnp.sum(v, axis=0)
            sq_cw = jnp.sum(v * v, axis=0)
            stats_ref[N_DEV - 1, 2 * bi] = jnp.sum(s_cw, axis=1, keepdims=True)
            stats_ref[N_DEV - 1, 2 * bi + 1] = jnp.sum(
                sq_cw, axis=1, keepdims=True
            )

        pl.semaphore_wait(barrier_sem, N_DEV - 1)
        sends = []
        for d in (1, 2, 3):
            rdma = pltpu.make_async_remote_copy(
                src_ref=stats_ref.at[N_DEV - 1],
                dst_ref=stats_ref.at[d - 1],
                send_sem=send_sems.at[d - 1],
                recv_sem=recv_sems.at[d - 1],
                device_id=((my + d) % N_DEV,),
                device_id_type=pl.DeviceIdType.MESH,
            )
            rdma.start()
            sends.append(rdma)
        for d in (1, 2, 3):
            recv = pltpu.make_async_remote_copy(
                src_ref=stats_ref.at[N_DEV - 1],
                dst_ref=stats_ref.at[d - 1],
                send_sem=send_sems.at[d - 1],
                recv_sem=recv_sems.at[d - 1],
                device_id=((my + d) % N_DEV,),
                device_id_type=pl.DeviceIdType.MESH,
            )
            recv.wait_recv()
        for rdma in sends:
            rdma.wait_send()

        eps = 1e-5
        inv_n = 1.0 / float(n_global)
        for bi in range(b):
            ssum = (
                stats_ref[0, 2 * bi]
                + stats_ref[1, 2 * bi]
                + stats_ref[2, 2 * bi]
                + stats_ref[3, 2 * bi]
            )
            ssq = (
                stats_ref[0, 2 * bi + 1]
                + stats_ref[1, 2 * bi + 1]
                + stats_ref[2, 2 * bi + 1]
                + stats_ref[3, 2 * bi + 1]
            )
            mean = ssum * inv_n
            var = ssq * inv_n - mean * mean
            st_ref[2 * bi] = mean
            st_ref[2 * bi + 1] = lax.rsqrt(var + eps)

    st = pl.pallas_call(
        stats_body,
        out_shape=jax.ShapeDtypeStruct((2 * b, c, 1), jnp.float32),
        in_specs=[pl.BlockSpec(memory_space=pltpu.MemorySpace.VMEM)],
        out_specs=pl.BlockSpec(memory_space=pltpu.MemorySpace.VMEM),
        scratch_shapes=[
            pltpu.VMEM((N_DEV, 2 * b, c, 1), jnp.float32),
            pltpu.SemaphoreType.DMA((N_DEV - 1,)),
            pltpu.SemaphoreType.DMA((N_DEV - 1,)),
        ],
        compiler_params=pltpu.CompilerParams(collective_id=0),
    )(xt)

    def main_body(x_ref, wp_ref, st_ref, out_hbm, outbuf, out_sems):
        wp = wp_ref[:, :]
        out_waits = [None, None]
        k = 0
        for bi in range(b):
            mean = st_ref[2 * bi][None, :, :]
            scale = st_ref[2 * bi + 1][None, :, :]
            for ci in range(NCHUNK):
                slot = k % 2
                if out_waits[slot] is not None:
                    out_waits[slot].wait()
                v = x_ref[bi, pl.ds(ci * hc, hc), :, :]
                hh = (v - mean) * scale
                a = hh * jax.nn.sigmoid(hh)
                res = lax.dot_general(
                    a,
                    wp,
                    dimension_numbers=(((1,), (0,)), ((), ())),
                    preferred_element_type=jnp.float32,
                )
                outbuf[slot] = res.reshape(hc * w, cout)
                cp = pltpu.make_async_copy(
                    outbuf.at[slot],
                    out_hbm.at[bi, pl.ds(ci * hc * w, hc * w), :],
                    out_sems.at[slot],
                )
                cp.start()
                out_waits[slot] = cp
                k += 1
        out_waits[0].wait()
        out_waits[1].wait()

    out = pl.pallas_call(
        main_body,
        out_shape=jax.ShapeDtypeStruct((b, h * w, cout), jnp.float32),
        in_specs=[
            pl.BlockSpec(memory_space=pltpu.MemorySpace.VMEM),
            pl.BlockSpec(memory_space=pltpu.MemorySpace.VMEM),
            pl.BlockSpec(memory_space=pltpu.MemorySpace.VMEM),
        ],
        out_specs=pl.BlockSpec(memory_space=pltpu.MemorySpace.HBM),
        scratch_shapes=[
            pltpu.VMEM((2, hc * w, cout), jnp.float32),
            pltpu.SemaphoreType.DMA((2,)),
        ],
    )(xt, Wp, st)
    return out.reshape(b, h, w, cout)
